# baseline (speedup 1.0000x reference)
import functools

import jax
import jax.numpy as jnp
from jax import lax
from jax.experimental import pallas as pl
from jax.experimental.pallas import tpu as pltpu
from jax.experimental.pallas import tpu_sc as plsc

F = 26
V = 100000
D = 32
B = 4096
EPS = 1e-5

NC = 2
NS = 16
NW = NC * NS
KPW = (F * D) // NW
NB16 = B // 16
H0 = 50048
H1 = V - H0


def _rsqrt(x):
    i = plsc.bitcast(x, jnp.int32)
    i = jnp.int32(0x5F3759DF) - (i >> 1)
    y = plsc.bitcast(i, jnp.float32)
    for _ in range(3):
        y = y * (1.5 - 0.5 * x * y * y)
    return y


@functools.partial(
    pl.kernel,
    out_type=(
        jax.ShapeDtypeStruct((F, D, B), jnp.float32),
        jax.ShapeDtypeStruct((NW, 2, 2, B), jnp.float32),
    ),
    mesh=plsc.VectorSubcoreMesh(
        core_axis_name="c", subcore_axis_name="s", num_cores=NC, num_subcores=NS
    ),
    compiler_params=pltpu.CompilerParams(
        use_tc_tiling_on_sc=True, needs_layout_passes=False
    ),
    scratch_types=[
        pltpu.VMEM((H0,), jnp.float32),
        pltpu.VMEM((H1,), jnp.float32),
        pltpu.VMEM((B,), jnp.int32),
        pltpu.VMEM((B,), jnp.float32),
        pltpu.VMEM((2, B), jnp.float32),
        pltpu.VMEM((2, B), jnp.float32),
        pltpu.VMEM((D,), jnp.float32),
        pltpu.VMEM((D,), jnp.float32),
        pltpu.SemaphoreType.DMA,
        pltpu.SemaphoreType.DMA,
    ],
)
def _sc_embed_ln(xt_hbm, tab_hbm, gamma_hbm, beta_hbm, out_hbm, part_hbm,
                 win0_v, win1_v, xcol_v, col_v, acc_v, acc2_v, g_v, b_v,
                 sem0, sem1):
    cid = lax.axis_index("c")
    sid = lax.axis_index("s")
    wid = cid * NS + sid
    f0 = (wid * KPW) // D

    zeros = jnp.zeros((16,), jnp.float32)

    def zero_acc(j, carry):
        s = pl.ds(j * 16, 16)
        acc_v[0, s] = zeros
        acc_v[1, s] = zeros
        acc2_v[0, s] = zeros
        acc2_v[1, s] = zeros
        return carry

    lax.fori_loop(0, NB16, zero_acc, 0)

    def _fd(k):
        w = wid * KPW + k
        return w // D, w % D

    def fire0(k):
        fp, dp = _fd(k)
        pltpu.async_copy(tab_hbm.at[fp, dp, pl.ds(0, H0)], win0_v, sem0)

    def fire1(k):
        fp, dp = _fd(k)
        pltpu.async_copy(tab_hbm.at[fp, dp, pl.ds(H0, H1)], win1_v, sem1)

    def wait0(k):
        fp, dp = _fd(k)
        pltpu.make_async_copy(
            tab_hbm.at[fp, dp, pl.ds(0, H0)], win0_v, sem0
        ).wait()

    def wait1(k):
        fp, dp = _fd(k)
        pltpu.make_async_copy(
            tab_hbm.at[fp, dp, pl.ds(H0, H1)], win1_v, sem1
        ).wait()

    pltpu.sync_copy(xt_hbm.at[f0], xcol_v)
    fire0(0)
    fire1(0)

    def pass1(k, fprev):
        fp, dp = _fd(k)
        slot = fp - f0

        @pl.when(fp != fprev)
        def _reload():
            pltpu.sync_copy(xt_hbm.at[fp], xcol_v)

        wait0(k)

        def inner0(j, c):
            for u in range(4):
                s = pl.ds(j * 64 + u * 16, 16)
                idx = xcol_v[s]
                m = idx < H0
                w = jnp.where(m, plsc.load_gather(win0_v, [idx], mask=m), 0.0)
                acc_v[slot, s] = acc_v[slot, s] + w
                acc2_v[slot, s] = acc2_v[slot, s] + w * w
                col_v[s] = w
            return c

        lax.fori_loop(0, NB16 // 4, inner0, 0)

        @pl.when(k < KPW - 1)
        def _prefetch0():
            fire0(k + 1)

        wait1(k)

        def inner1(j, c):
            for u in range(4):
                s = pl.ds(j * 64 + u * 16, 16)
                idx = xcol_v[s]
                m = idx >= H0
                w = jnp.where(
                    m, plsc.load_gather(win1_v, [idx - H0], mask=m), 0.0
                )
                acc_v[slot, s] = acc_v[slot, s] + w
                acc2_v[slot, s] = acc2_v[slot, s] + w * w
                col_v[s] = col_v[s] + w
            return c

        lax.fori_loop(0, NB16 // 4, inner1, 0)

        @pl.when(k < KPW - 1)
        def _prefetch1():
            fire1(k + 1)

        pltpu.sync_copy(col_v, out_hbm.at[fp, dp])
        return fp

    lax.fori_loop(0, KPW, pass1, f0)

    pltpu.sync_copy(acc_v, part_hbm.at[wid, 0])
    pltpu.sync_copy(acc2_v, part_hbm.at[wid, 1])
    plsc.subcore_barrier()

    @pl.when(sid < F // NC)
    def _owner():
        f = cid * (F // NC) + sid
        t0 = (f * D) // KPW

        def zero_own(j, carry):
            s = pl.ds(j * 16, 16)
            acc_v[0, s] = zeros
            acc2_v[0, s] = zeros
            return carry

        lax.fori_loop(0, NB16, zero_own, 0)

        for dt in range(3):
            t = t0 + dt
            fl = (t * KPW) // D
            sl = f - fl
            valid = (t < NW) & (t * KPW < f * D + D) & (sl >= 0) & (sl < 2)

            @pl.when(valid)
            def _combine():
                tc = jnp.minimum(t, NW - 1)
                slc = jnp.clip(sl, 0, 1)
                pltpu.sync_copy(part_hbm.at[tc, 0, slc], col_v)

                def add_m(j, c):
                    s = pl.ds(j * 16, 16)
                    acc_v[0, s] = acc_v[0, s] + col_v[s]
                    return c

                lax.fori_loop(0, NB16, add_m, 0)
                pltpu.sync_copy(part_hbm.at[tc, 1, slc], col_v)

                def add_m2(j, c):
                    s = pl.ds(j * 16, 16)
                    acc2_v[0, s] = acc2_v[0, s] + col_v[s]
                    return c

                lax.fori_loop(0, NB16, add_m2, 0)

        pltpu.sync_copy(gamma_hbm, g_v)
        pltpu.sync_copy(beta_hbm, b_v)

        def fin(j, carry):
            s = pl.ds(j * 16, 16)
            m = acc_v[0, s] * (1.0 / D)
            var = acc2_v[0, s] * (1.0 / D) - m * m
            acc_v[0, s] = m
            acc2_v[0, s] = _rsqrt(var + EPS)
            return carry

        lax.fori_loop(0, NB16, fin, 0)

        colA = win0_v.at[pl.ds(0, B)]
        colB = win1_v.at[pl.ds(0, B)]

        def rd(d, buf, sem):
            pltpu.async_copy(out_hbm.at[f, d], buf, sem)

        def wr(d, buf, sem):
            pltpu.async_copy(buf, out_hbm.at[f, d], sem)

        def wt(buf, sem):
            pltpu.make_async_copy(out_hbm.at[f, 0], buf, sem).wait()

        def normalize(buf, d):
            dcol = jnp.zeros((16,), jnp.int32) + d
            g = plsc.load_gather(g_v, [dcol])
            b = plsc.load_gather(b_v, [dcol])

            def inner(j, c):
                for u in range(4):
                    s = pl.ds(j * 64 + u * 16, 16)
                    buf[s] = (buf[s] - acc_v[0, s]) * acc2_v[0, s] * g + b
                return c

            lax.fori_loop(0, NB16 // 4, inner, 0)

        rd(0, colA, sem0)
        rd(1, colB, sem1)

        def pass2(i, carry):
            d0 = i * 2
            wt(colA, sem0)
            normalize(colA, d0)
            wr(d0, colA, sem0)
            wt(colB, sem1)
            normalize(colB, d0 + 1)
            wr(d0 + 1, colB, sem1)

            @pl.when(i < D // 2 - 1)
            def _next():
                wt(colA, sem0)
                rd(d0 + 2, colA, sem0)
                wt(colB, sem1)
                rd(d0 + 3, colB, sem1)

            return carry

        lax.fori_loop(0, D // 2, pass2, 0)
        wt(colA, sem0)
        wt(colB, sem1)


def kernel(x, tables, gamma, beta):
    xt = x.T.astype(jnp.int32)
    tab_t = jnp.transpose(tables, (0, 2, 1))
    out, _ = _sc_embed_ln(xt, tab_t,
                          gamma.astype(jnp.float32), beta.astype(jnp.float32))
    return jnp.transpose(out, (2, 0, 1))


# speedup vs baseline: 1.7073x; 1.7073x over previous
import functools

import jax
import jax.numpy as jnp
from jax import lax
from jax.experimental import pallas as pl
from jax.experimental.pallas import tpu as pltpu
from jax.experimental.pallas import tpu_sc as plsc

F = 26
V = 100000
D = 32
B = 4096
EPS = 1e-5

NC = 2
NS = 16
NW = NC * NS
KPW = (F * D) // NW
NB16 = B // 16
H0 = 50048
H1 = V - H0


def _rsqrt(x):
    i = plsc.bitcast(x, jnp.int32)
    i = jnp.int32(0x5F3759DF) - (i >> 1)
    y = plsc.bitcast(i, jnp.float32)
    for _ in range(3):
        y = y * (1.5 - 0.5 * x * y * y)
    return y


@functools.partial(
    pl.kernel,
    out_type=(
        jax.ShapeDtypeStruct((F, D, B), jnp.float32),
        jax.ShapeDtypeStruct((NW, 2, 2, B), jnp.float32),
    ),
    mesh=plsc.VectorSubcoreMesh(
        core_axis_name="c", subcore_axis_name="s", num_cores=NC, num_subcores=NS
    ),
    compiler_params=pltpu.CompilerParams(
        use_tc_tiling_on_sc=True, needs_layout_passes=False
    ),
    scratch_types=[
        pltpu.VMEM((H0,), jnp.float32),
        pltpu.VMEM((H1,), jnp.float32),
        pltpu.VMEM((B,), jnp.int32),
        pltpu.VMEM((B,), jnp.float32),
        pltpu.VMEM((B,), jnp.float32),
        pltpu.VMEM((B,), jnp.float32),
        pltpu.VMEM((D,), jnp.float32),
        pltpu.VMEM((D,), jnp.float32),
        pltpu.SemaphoreType.DMA,
        pltpu.SemaphoreType.DMA,
    ],
)
def _sc_embed_ln(xt_hbm, tab_hbm, gamma_hbm, beta_hbm, out_hbm, part_hbm,
                 win0_v, win1_v, xcol_v, col_v, acc_v, acc2_v, g_v, b_v,
                 sem0, sem1):
    cid = lax.axis_index("c")
    sid = lax.axis_index("s")
    wid = cid * NS + sid
    f0 = (wid * KPW) // D

    zeros = jnp.zeros((16,), jnp.float32)

    def zero_acc(j, carry):
        s = pl.ds(j * 16, 16)
        acc_v[s] = zeros
        acc2_v[s] = zeros
        return carry

    lax.fori_loop(0, NB16, zero_acc, 0)

    def publish(fprev):
        sl = fprev - f0
        pltpu.sync_copy(acc_v, part_hbm.at[wid, 0, sl])
        pltpu.sync_copy(acc2_v, part_hbm.at[wid, 1, sl])

    def _fd(k):
        w = wid * KPW + k
        return w // D, w % D

    def fire0(k):
        fp, dp = _fd(k)
        pltpu.async_copy(tab_hbm.at[fp, dp, pl.ds(0, H0)], win0_v, sem0)

    def fire1(k):
        fp, dp = _fd(k)
        pltpu.async_copy(tab_hbm.at[fp, dp, pl.ds(H0, H1)], win1_v, sem1)

    def wait0(k):
        fp, dp = _fd(k)
        pltpu.make_async_copy(
            tab_hbm.at[fp, dp, pl.ds(0, H0)], win0_v, sem0
        ).wait()

    def wait1(k):
        fp, dp = _fd(k)
        pltpu.make_async_copy(
            tab_hbm.at[fp, dp, pl.ds(H0, H1)], win1_v, sem1
        ).wait()

    pltpu.sync_copy(xt_hbm.at[f0], xcol_v)
    fire0(0)
    fire1(0)

    def pass1(k, fprev):
        fp, dp = _fd(k)

        @pl.when(fp != fprev)
        def _switch():
            publish(fprev)
            pltpu.sync_copy(xt_hbm.at[fp], xcol_v)

            def zero_again(j, c):
                s = pl.ds(j * 16, 16)
                acc_v[s] = zeros
                acc2_v[s] = zeros
                return c

            lax.fori_loop(0, NB16, zero_again, 0)

        wait0(k)

        def inner0(j, c):
            for u in range(4):
                s = pl.ds(j * 64 + u * 16, 16)
                idx = xcol_v[s]
                m = idx < H0
                w = jnp.where(m, plsc.load_gather(win0_v, [idx], mask=m), 0.0)
                acc_v[s] = acc_v[s] + w
                acc2_v[s] = acc2_v[s] + w * w
                col_v[s] = w
            return c

        lax.fori_loop(0, NB16 // 4, inner0, 0)

        @pl.when(k < KPW - 1)
        def _prefetch0():
            fire0(k + 1)

        wait1(k)

        def inner1(j, c):
            for u in range(4):
                s = pl.ds(j * 64 + u * 16, 16)
                idx = xcol_v[s]
                m = idx >= H0
                w = jnp.where(
                    m, plsc.load_gather(win1_v, [idx - H0], mask=m), 0.0
                )
                acc_v[s] = acc_v[s] + w
                acc2_v[s] = acc2_v[s] + w * w
                col_v[s] = col_v[s] + w
            return c

        lax.fori_loop(0, NB16 // 4, inner1, 0)

        @pl.when(k < KPW - 1)
        def _prefetch1():
            fire1(k + 1)

        pltpu.sync_copy(col_v, out_hbm.at[fp, dp])
        return fp

    flast = lax.fori_loop(0, KPW, pass1, f0)

    publish(flast)
    plsc.subcore_barrier()

    @pl.when(sid < F // NC)
    def _owner():
        f = cid * (F // NC) + sid
        t0 = (f * D) // KPW

        def zero_own(j, carry):
            s = pl.ds(j * 16, 16)
            acc_v[s] = zeros
            acc2_v[s] = zeros
            return carry

        lax.fori_loop(0, NB16, zero_own, 0)

        for dt in range(3):
            t = t0 + dt
            fl = (t * KPW) // D
            sl = f - fl
            valid = (t < NW) & (t * KPW < f * D + D) & (sl >= 0) & (sl < 2)

            @pl.when(valid)
            def _combine():
                tc = jnp.minimum(t, NW - 1)
                slc = jnp.clip(sl, 0, 1)
                pltpu.sync_copy(part_hbm.at[tc, 0, slc], col_v)

                def add_m(j, c):
                    s = pl.ds(j * 16, 16)
                    acc_v[s] = acc_v[s] + col_v[s]
                    return c

                lax.fori_loop(0, NB16, add_m, 0)
                pltpu.sync_copy(part_hbm.at[tc, 1, slc], col_v)

                def add_m2(j, c):
                    s = pl.ds(j * 16, 16)
                    acc2_v[s] = acc2_v[s] + col_v[s]
                    return c

                lax.fori_loop(0, NB16, add_m2, 0)

        pltpu.sync_copy(gamma_hbm, g_v)
        pltpu.sync_copy(beta_hbm, b_v)

        def fin(j, carry):
            s = pl.ds(j * 16, 16)
            m = acc_v[s] * (1.0 / D)
            var = acc2_v[s] * (1.0 / D) - m * m
            acc_v[s] = m
            acc2_v[s] = _rsqrt(var + EPS)
            return carry

        lax.fori_loop(0, NB16, fin, 0)

        colA = win0_v.at[pl.ds(0, B)]
        colB = win1_v.at[pl.ds(0, B)]

        def rd(d, buf, sem):
            pltpu.async_copy(out_hbm.at[f, d], buf, sem)

        def wr(d, buf, sem):
            pltpu.async_copy(buf, out_hbm.at[f, d], sem)

        def wt(buf, sem):
            pltpu.make_async_copy(out_hbm.at[f, 0], buf, sem).wait()

        def normalize(buf, d):
            dcol = jnp.zeros((16,), jnp.int32) + d
            g = plsc.load_gather(g_v, [dcol])
            b = plsc.load_gather(b_v, [dcol])

            def inner(j, c):
                for u in range(4):
                    s = pl.ds(j * 64 + u * 16, 16)
                    buf[s] = (buf[s] - acc_v[s]) * acc2_v[s] * g + b
                return c

            lax.fori_loop(0, NB16 // 4, inner, 0)

        rd(0, colA, sem0)
        rd(1, colB, sem1)

        def pass2(i, carry):
            d0 = i * 2
            wt(colA, sem0)
            normalize(colA, d0)
            wr(d0, colA, sem0)
            wt(colB, sem1)
            normalize(colB, d0 + 1)
            wr(d0 + 1, colB, sem1)

            @pl.when(i < D // 2 - 1)
            def _next():
                wt(colA, sem0)
                rd(d0 + 2, colA, sem0)
                wt(colB, sem1)
                rd(d0 + 3, colB, sem1)

            return carry

        lax.fori_loop(0, D // 2, pass2, 0)
        wt(colA, sem0)
        wt(colB, sem1)


def kernel(x, tables, gamma, beta):
    xt = x.T.astype(jnp.int32)
    tab_t = jnp.transpose(tables, (0, 2, 1))
    out, _ = _sc_embed_ln(xt, tab_t,
                          gamma.astype(jnp.float32), beta.astype(jnp.float32))
    return jnp.transpose(out, (2, 0, 1))
